# Initial kernel scaffold; baseline (speedup 1.0000x reference)
import jax
import jax.numpy as jnp
from jax.experimental import pallas as pl


def kernel(token_positions, pe):
    raise NotImplementedError("write your pallas kernel here")


# speedup vs baseline: 1.9890x; 1.9890x over previous
import functools

import jax
import jax.numpy as jnp
from jax import lax
from jax.experimental import pallas as pl
from jax.experimental.pallas import tpu as pltpu
from jax.experimental.pallas import tpu_sc as plsc

D_MODEL = 1024
N_TOKENS = 4 * 8192

_info = plsc.get_sparse_core_info()
_NC, _NS = _info.num_cores, _info.num_subcores
_NW = _NC * _NS
_B_PER_W = N_TOKENS // _NW
_CHUNK = 32
_N_CHUNKS = _B_PER_W // _CHUNK


def _gather_body(idx_hbm, pe_hbm, out_hbm, idx_v, rows_v, gsem):
    wid = lax.axis_index("s") * _NC + lax.axis_index("c")
    base = wid * _B_PER_W
    pltpu.sync_copy(idx_hbm.at[pl.ds(base, _B_PER_W)], idx_v)

    def chunk(c, carry):
        off = pl.multiple_of(c * _CHUNK, _CHUNK)
        pltpu.async_copy(
            pe_hbm.at[idx_v.at[pl.ds(off, _CHUNK)]], rows_v, gsem
        ).wait()
        pltpu.sync_copy(rows_v, out_hbm.at[pl.ds(base + off, _CHUNK)])
        return carry

    lax.fori_loop(0, _N_CHUNKS, chunk, 0)


@jax.jit
def kernel(token_positions, pe):
    idx = token_positions.reshape(N_TOKENS).astype(jnp.int32)
    out = pl.kernel(
        _gather_body,
        out_type=jax.ShapeDtypeStruct((N_TOKENS, D_MODEL), jnp.float32),
        mesh=plsc.VectorSubcoreMesh(core_axis_name="c", subcore_axis_name="s"),
        scratch_types=[
            pltpu.VMEM((_B_PER_W,), jnp.int32),
            pltpu.VMEM((_CHUNK, D_MODEL), jnp.float32),
            pltpu.SemaphoreType.DMA,
        ],
    )(idx, pe)
    return out.reshape(token_positions.shape + (D_MODEL,))


# speedup vs baseline: 2.3089x; 1.1608x over previous
import jax
import jax.numpy as jnp
from jax import lax
from jax.experimental import pallas as pl
from jax.experimental.pallas import tpu as pltpu
from jax.experimental.pallas import tpu_sc as plsc

D_MODEL = 1024
N_TOKENS = 4 * 8192

_info = plsc.get_sparse_core_info()
_NC, _NS = _info.num_cores, _info.num_subcores
_NW = _NC * _NS
_B_PER_W = N_TOKENS // _NW
_CHUNK = 32
_N_CHUNKS = _B_PER_W // _CHUNK


def _gather_body(idx_hbm, pe_hbm, out_hbm, idx_v, rows0, rows1,
                 gs0, gs1, os0, os1):
    wid = lax.axis_index("s") * _NC + lax.axis_index("c")
    base = wid * _B_PER_W
    pltpu.sync_copy(idx_hbm.at[pl.ds(base, _B_PER_W)], idx_v)

    rows = (rows0, rows1)
    gs = (gs0, gs1)
    os = (os0, os1)

    def start_gather(c, b):
        off = pl.multiple_of(c * _CHUNK, _CHUNK)
        pltpu.async_copy(pe_hbm.at[idx_v.at[pl.ds(off, _CHUNK)]],
                         rows[b], gs[b])

    def wait_gather(b):
        pltpu.make_async_copy(pe_hbm.at[idx_v.at[pl.ds(0, _CHUNK)]],
                              rows[b], gs[b]).wait()

    def start_write(c, b):
        off = pl.multiple_of(c * _CHUNK, _CHUNK)
        pltpu.async_copy(rows[b], out_hbm.at[pl.ds(base + off, _CHUNK)],
                         os[b])

    def wait_write(b):
        pltpu.make_async_copy(rows[b], out_hbm.at[pl.ds(base, _CHUNK)],
                              os[b]).wait()

    start_gather(0, 0)
    wait_gather(0)
    start_gather(1, 1)
    start_write(0, 0)

    def pair(i, carry):
        k = 1 + 2 * i
        wait_gather(1)
        wait_write(0)
        start_gather(k + 1, 0)
        start_write(k, 1)
        wait_gather(0)
        wait_write(1)
        start_gather(k + 2, 1)
        start_write(k + 1, 0)
        return carry

    lax.fori_loop(0, (_N_CHUNKS - 2) // 2, pair, 0)

    wait_gather(1)
    wait_write(0)
    start_write(_N_CHUNKS - 1, 1)
    wait_write(1)


@jax.jit
def kernel(token_positions, pe):
    idx = token_positions.reshape(N_TOKENS).astype(jnp.int32)
    out = pl.kernel(
        _gather_body,
        out_type=jax.ShapeDtypeStruct((N_TOKENS, D_MODEL), jnp.float32),
        mesh=plsc.VectorSubcoreMesh(core_axis_name="c", subcore_axis_name="s"),
        scratch_types=[
            pltpu.VMEM((_B_PER_W,), jnp.int32),
            pltpu.VMEM((_CHUNK, D_MODEL), jnp.float32),
            pltpu.VMEM((_CHUNK, D_MODEL), jnp.float32),
            pltpu.SemaphoreType.DMA,
            pltpu.SemaphoreType.DMA,
            pltpu.SemaphoreType.DMA,
            pltpu.SemaphoreType.DMA,
        ],
    )(idx, pe)
    return out.reshape(token_positions.shape + (D_MODEL,))


# speedup vs baseline: 2.3705x; 1.0267x over previous
import jax
import jax.numpy as jnp
from jax import lax
from jax.experimental import pallas as pl
from jax.experimental.pallas import tpu as pltpu
from jax.experimental.pallas import tpu_sc as plsc

D_MODEL = 1024
N_TOKENS = 4 * 8192

_info = plsc.get_sparse_core_info()
_NC, _NS = _info.num_cores, _info.num_subcores
_NW = _NC * _NS
_B_PER_W = N_TOKENS // _NW
_CHUNK = 32
_N_CHUNKS = _B_PER_W // _CHUNK


def _gather_body(idx_hbm, pe_hbm, out_hbm, idx_v, rows0, rows1, rows2,
                 gs0, gs1, gs2, os0, os1, os2):
    wid = lax.axis_index("s") * _NC + lax.axis_index("c")
    base = wid * _B_PER_W
    pltpu.sync_copy(idx_hbm.at[pl.ds(base, _B_PER_W)], idx_v)

    rows = (rows0, rows1, rows2)
    gs = (gs0, gs1, gs2)
    os = (os0, os1, os2)

    def start_gather(c, b):
        off = pl.multiple_of(c * _CHUNK, _CHUNK)
        pltpu.async_copy(pe_hbm.at[idx_v.at[pl.ds(off, _CHUNK)]],
                         rows[b], gs[b])

    def wait_gather(b):
        pltpu.make_async_copy(pe_hbm.at[idx_v.at[pl.ds(0, _CHUNK)]],
                              rows[b], gs[b]).wait()

    def start_write(c, b):
        off = pl.multiple_of(c * _CHUNK, _CHUNK)
        pltpu.async_copy(rows[b], out_hbm.at[pl.ds(base + off, _CHUNK)],
                         os[b])

    def wait_write(b):
        pltpu.make_async_copy(rows[b], out_hbm.at[pl.ds(base, _CHUNK)],
                              os[b]).wait()

    def step(k, b, nb, start_next):
        wait_gather(b)
        start_write(k, b)
        if start_next:
            wait_write(nb)
            start_gather(k + 2, nb)

    start_gather(0, 0)
    start_gather(1, 1)
    wait_gather(0)
    start_write(0, 0)
    start_gather(2, 2)

    def triple(i, carry):
        k = 1 + 3 * i
        step(k, 1, 0, True)
        step(k + 1, 2, 1, True)
        step(k + 2, 0, 2, True)
        return carry

    lax.fori_loop(0, (_N_CHUNKS - 4) // 3, triple, 0)

    step(_N_CHUNKS - 4, 1, 0, True)
    step(_N_CHUNKS - 3, 2, 1, True)
    step(_N_CHUNKS - 2, 0, 2, False)
    step(_N_CHUNKS - 1, 1, 0, False)
    wait_write(2)
    wait_write(0)
    wait_write(1)


@jax.jit
def kernel(token_positions, pe):
    idx = token_positions.reshape(N_TOKENS).astype(jnp.int32)
    out = pl.kernel(
        _gather_body,
        out_type=jax.ShapeDtypeStruct((N_TOKENS, D_MODEL), jnp.float32),
        mesh=plsc.VectorSubcoreMesh(core_axis_name="c", subcore_axis_name="s"),
        scratch_types=[
            pltpu.VMEM((_B_PER_W,), jnp.int32),
            pltpu.VMEM((_CHUNK, D_MODEL), jnp.float32),
            pltpu.VMEM((_CHUNK, D_MODEL), jnp.float32),
            pltpu.VMEM((_CHUNK, D_MODEL), jnp.float32),
            pltpu.SemaphoreType.DMA,
            pltpu.SemaphoreType.DMA,
            pltpu.SemaphoreType.DMA,
            pltpu.SemaphoreType.DMA,
            pltpu.SemaphoreType.DMA,
            pltpu.SemaphoreType.DMA,
        ],
    )(idx, pe)
    return out.reshape(token_positions.shape + (D_MODEL,))
